# baseline (speedup 1.0000x reference)
import jax
import jax.numpy as jnp
from jax import lax
from jax.experimental import pallas as pl
from jax.experimental.pallas import tpu as pltpu
from jax.experimental.pallas import tpu_sc as plsc

_B = 16384
_D = 128
_ITEM_NUM = 100000
_NW = 32
_RPW = _B // _NW
_CH = 128


def _rng_consts():
    with jax.ensure_compile_time_eval():
        kk = jax.random.key(42)
        k1, k2 = jax.random.split(kk)
        neg = jax.random.randint(k1, (_B, 4), 0, _ITEM_NUM)
        delta = jnp.clip(
            jax.random.normal(k2, (_B, _D), jnp.float32) * 0.1 + 0.5, 0.0, 1.0)
        neg2d = neg.reshape(_B * 4 // 128, 128).astype(jnp.int32)
    return neg2d, delta


def _sc_main_body(u_emb, v_emb, nu2d, nv2d, neg2d, delta,
                  u_id_o, v_id_o,
                  idx_u, idx_v, idx_n, r0, r1, r2, r3, potc, dbuf,
                  gs0, gs1, gs2, gs3, ws0, ws1, ws2, ws3, dsem):
    wid = lax.axis_index("s") * 2 + lax.axis_index("c")
    base = wid * _RPW

    ring = (r0, r1, r2, r3)
    gsem = (gs0, gs1, gs2, gs3)
    wsem = (ws0, ws1, ws2, ws3)

    pltpu.sync_copy(nu2d.at[pl.ds(4 * wid, 4)], idx_u)
    pltpu.sync_copy(nv2d.at[pl.ds(4 * wid, 4)], idx_v)
    pltpu.sync_copy(neg2d.at[pl.ds(16 * wid, 16)], idx_n)

    jobs = []
    for c in range(4):
        for m in range(4):
            jobs.append(("neg", v_emb, idx_n, 4 * c + m, None, 32 * m))
        jobs.append(("plain", u_emb, idx_u, c, u_id_o, base + _CH * c))
        jobs.append(("vblend", v_emb, idx_v, c, v_id_o, base + _CH * c))

    g = [None, None, None, None]
    w = [None, None, None, None]
    dcp = [None]

    def pool(src, off):
        @plsc.parallel_loop(0, 32, unroll=2)
        def _row(r):
            for j in range(_D // 16):
                sl = pl.ds(16 * j, 16)
                s = ((src[4 * r, sl] + src[4 * r + 1, sl])
                     + (src[4 * r + 2, sl] + src[4 * r + 3, sl]))
                potc[off + r, sl] = s * 0.25

    def blend(src):
        @plsc.parallel_loop(0, _CH, unroll=4)
        def _row(r):
            for j in range(_D // 16):
                sl = pl.ds(16 * j, 16)
                p = potc[r, sl]
                src[r, sl] = p + dbuf[r, sl] * (src[r, sl] - p)

    for i in range(len(jobs) + 3):
        if i < len(jobs):
            s = i % 4
            if i >= 4 and w[s] is not None:
                w[s].wait()
                w[s] = None
            kind, tbl, idx, row, out, off = jobs[i]
            g[s] = pltpu.async_copy(tbl.at[idx.at[row]], ring[s], gsem[s])
            if kind == "vblend":
                dcp[0] = pltpu.async_copy(delta.at[pl.ds(off, _CH)], dbuf, dsem)
        p = i - 3
        if 0 <= p < len(jobs):
            sp = p % 4
            g[sp].wait()
            kind, tbl, idx, row, out, off = jobs[p]
            if kind == "neg":
                pool(ring[sp], off)
                w[sp] = None
            elif kind == "vblend":
                dcp[0].wait()
                blend(ring[sp])
                w[sp] = pltpu.async_copy(ring[sp], out.at[pl.ds(off, _CH)],
                                         wsem[sp])
            else:
                w[sp] = pltpu.async_copy(ring[sp], out.at[pl.ds(off, _CH)],
                                         wsem[sp])
    for s in range(4):
        if w[s] is not None:
            w[s].wait()


def _sc_rev_body(u_rev, v_rev, nu2d, nv2d, u_rev_o, v_rev_o,
                 idx_u, idx_v, r0, r1, r2, r3,
                 gs0, gs1, gs2, gs3, ws0, ws1, ws2, ws3):
    wid = lax.axis_index("s") * 2 + lax.axis_index("c")
    base = wid * _RPW

    ring = (r0, r1, r2, r3)
    gsem = (gs0, gs1, gs2, gs3)
    wsem = (ws0, ws1, ws2, ws3)

    pltpu.sync_copy(nu2d.at[pl.ds(4 * wid, 4)], idx_u)
    pltpu.sync_copy(nv2d.at[pl.ds(4 * wid, 4)], idx_v)

    jobs = []
    for c in range(4):
        jobs.append((u_rev, idx_u, c, u_rev_o, base + _CH * c))
        jobs.append((v_rev, idx_v, c, v_rev_o, base + _CH * c))

    g = [None, None, None, None]
    w = [None, None, None, None]
    for i in range(len(jobs) + 3):
        if i < len(jobs):
            s = i % 4
            if w[s] is not None:
                w[s].wait()
                w[s] = None
            tbl, idx, row, out, off = jobs[i]
            g[s] = pltpu.async_copy(tbl.at[idx.at[row]], ring[s], gsem[s])
        p = i - 3
        if 0 <= p < len(jobs):
            sp = p % 4
            g[sp].wait()
            tbl, idx, row, out, off = jobs[p]
            w[sp] = pltpu.async_copy(ring[sp], out.at[pl.ds(off, _CH)],
                                     wsem[sp])
    for s in range(4):
        if w[s] is not None:
            w[s].wait()


def _sc_gather_main(u_emb, v_emb, nu2d, nv2d, neg2d, delta):
    f32 = jnp.float32
    out = jax.ShapeDtypeStruct((_B, _D), f32)
    run = pl.kernel(
        _sc_main_body,
        mesh=plsc.VectorSubcoreMesh(core_axis_name="c", subcore_axis_name="s"),
        out_type=[out, out],
        scratch_types=(
            [pltpu.VMEM((4, 128), jnp.int32),
             pltpu.VMEM((4, 128), jnp.int32),
             pltpu.VMEM((16, 128), jnp.int32)]
            + [pltpu.VMEM((_CH, _D), f32) for _ in range(4)]
            + [pltpu.VMEM((_CH, _D), f32),
               pltpu.VMEM((_CH, _D), f32)]
            + [pltpu.SemaphoreType.DMA for _ in range(9)]
        ),
    )
    return run(u_emb, v_emb, nu2d, nv2d, neg2d, delta)


def _sc_gather_rev(u_rev, v_rev, nu2d, nv2d):
    f32 = jnp.float32
    out = jax.ShapeDtypeStruct((_B, _D), f32)
    run = pl.kernel(
        _sc_rev_body,
        mesh=plsc.VectorSubcoreMesh(core_axis_name="c", subcore_axis_name="s"),
        out_type=[out, out],
        scratch_types=(
            [pltpu.VMEM((4, 128), jnp.int32),
             pltpu.VMEM((4, 128), jnp.int32)]
            + [pltpu.VMEM((_CH, _D), f32) for _ in range(4)]
            + [pltpu.SemaphoreType.DMA for _ in range(8)]
        ),
    )
    return run(u_rev, v_rev, nu2d, nv2d)


def _blend_body(vraw, pot, delta, out):
    p = pot[:]
    out[:] = p + delta[:] * (vraw[:] - p)


def _blend(vraw, pot, delta):
    blk = pl.BlockSpec((1024, _D), lambda i: (i, 0))
    return pl.pallas_call(
        _blend_body,
        grid=(_B // 1024,),
        in_specs=[blk, blk, blk],
        out_specs=blk,
        out_shape=jax.ShapeDtypeStruct((_B, _D), jnp.float32),
    )(vraw, pot, delta)


_MBLK = 1024
_NBLK = _B // _MBLK


def _mlp_body(u_ref, v_ref, w1u, w1v, b1, g1, be1, w2, b2, g2, be2,
              w3, b3, g3, be3, wc, bc, out_ref,
              z1s, z2s, z3s, s1, q1, s2, q2, s3, q3):
    eps = 1e-5
    inv_b = 1.0 / _B
    ph = pl.program_id(0)
    b = pl.program_id(1)

    def mm(x, w):
        return jnp.dot(x, w, preferred_element_type=jnp.float32)

    def acc(sref, qref, z):
        @pl.when(b == 0)
        def _():
            sref[:] = jnp.zeros_like(sref)
            qref[:] = jnp.zeros_like(qref)
        sref[:] += jnp.sum(z, axis=0, keepdims=True)
        qref[:] += jnp.sum(z * z, axis=0, keepdims=True)

    def norm(sref, qref, z, g, be):
        mu = sref[:] * inv_b
        var = qref[:] * inv_b - mu * mu
        return g[:] * (z - mu) * lax.rsqrt(var + eps) + be[:]

    @pl.when(ph == 0)
    def _():
        z1 = jnp.maximum(mm(u_ref[:], w1u[:]) + mm(v_ref[:], w1v[:]) + b1[:],
                         0.0)
        z1s[pl.ds(b * _MBLK, _MBLK), :] = z1
        acc(s1, q1, z1)

    @pl.when(ph == 1)
    def _():
        h1 = norm(s1, q1, z1s[pl.ds(b * _MBLK, _MBLK), :], g1, be1)
        z2 = jnp.maximum(mm(h1, w2[:]) + b2[:], 0.0)
        z2s[pl.ds(b * _MBLK, _MBLK), :] = z2
        acc(s2, q2, z2)

    @pl.when(ph == 2)
    def _():
        h2 = norm(s2, q2, z2s[pl.ds(b * _MBLK, _MBLK), :], g2, be2)
        z3 = jnp.maximum(mm(h2, w3[:]) + b3[:], 0.0)
        z3s[pl.ds(b * _MBLK, _MBLK), :] = z3
        acc(s3, q3, z3)

    @pl.when(ph == 3)
    def _():
        h3 = norm(s3, q3, z3s[pl.ds(b * _MBLK, _MBLK), :], g3, be3)
        logit = mm(h3, wc[:]) + bc[:]
        out_ref[:] = 1.0 / (1.0 + jnp.exp(-logit))


def _mlp(u_id, v_id, W1, b1, W2, b2, W3, b3, Wc, bc, g1, be1, g2, be2, g3, be3):
    f32 = jnp.float32
    io_spec = pl.BlockSpec(
        (_MBLK, _D), lambda ph, b: (jnp.where(ph == 0, b, 0), 0))
    full = lambda r, c: pl.BlockSpec((r, c), lambda ph, b: (0, 0))
    return pl.pallas_call(
        _mlp_body,
        grid=(4, _NBLK),
        in_specs=[io_spec, io_spec,
                  full(_D, _D), full(_D, _D), full(1, _D), full(1, _D),
                  full(1, _D), full(_D, _D // 2), full(1, _D // 2),
                  full(1, _D // 2), full(1, _D // 2), full(_D // 2, _D // 4),
                  full(1, _D // 4), full(1, _D // 4), full(1, _D // 4),
                  full(_D // 4, 1), full(1, 1)],
        out_specs=pl.BlockSpec((_MBLK, 1),
                               lambda ph, b: (jnp.where(ph == 3, b, 0), 0)),
        out_shape=jax.ShapeDtypeStruct((_B, 1), f32),
        scratch_shapes=[
            pltpu.VMEM((_B, _D), f32),
            pltpu.VMEM((_B, _D // 2), f32),
            pltpu.VMEM((_B, _D // 4), f32),
            pltpu.VMEM((1, _D), f32), pltpu.VMEM((1, _D), f32),
            pltpu.VMEM((1, _D // 2), f32), pltpu.VMEM((1, _D // 2), f32),
            pltpu.VMEM((1, _D // 4), f32), pltpu.VMEM((1, _D // 4), f32),
        ],
    )(u_id, v_id,
      W1[:, :_D].T, W1[:, _D:].T, b1.reshape(1, -1), g1.reshape(1, -1),
      be1.reshape(1, -1), W2.T, b2.reshape(1, -1), g2.reshape(1, -1),
      be2.reshape(1, -1), W3.T, b3.reshape(1, -1), g3.reshape(1, -1),
      be3.reshape(1, -1), Wc.T, bc.reshape(1, -1))


def kernel(nodes_u, nodes_v, global_protos, inter_nums, u_emb_w, v_emb_w,
           u_rev_w, v_rev_w, W1, b1, W2, b2, W3, b3, Wc, bc,
           g1, be1, g2, be2, g3, be3):
    neg2d, delta = _rng_consts()
    nu2d = nodes_u.astype(jnp.int32).reshape(_B // 128, 128)
    nv2d = nodes_v.astype(jnp.int32).reshape(_B // 128, 128)
    u_id, v_id = _sc_gather_main(u_emb_w, v_emb_w, nu2d, nv2d, neg2d, delta)
    u_review, v_review = _sc_gather_rev(u_rev_w, v_rev_w, nu2d, nv2d)
    pred = _mlp(u_id, v_id, W1, b1, W2, b2, W3, b3, Wc, bc,
                g1, be1, g2, be2, g3, be3)
    return (pred[:, 0], u_id, v_id, u_review, v_review)


# speedup vs baseline: 1.2221x; 1.2221x over previous
import jax
import jax.numpy as jnp
from jax import lax
from jax.experimental import pallas as pl
from jax.experimental.pallas import tpu as pltpu
from jax.experimental.pallas import tpu_sc as plsc

_B = 16384
_D = 128
_ITEM_NUM = 100000
_NW = 32
_RPW = _B // _NW
_CH = 128


def _rng_consts():
    with jax.ensure_compile_time_eval():
        kk = jax.random.key(42)
        k1, k2 = jax.random.split(kk)
        neg = jax.random.randint(k1, (_B, 4), 0, _ITEM_NUM)
        delta = jnp.clip(
            jax.random.normal(k2, (_B, _D), jnp.float32) * 0.1 + 0.5, 0.0, 1.0)
        neg2d = neg.reshape(_B * 4 // 128, 128).astype(jnp.int32)
    return neg2d, delta


def _sc_main_body(u_emb, v_emb, nu2d, nv2d, neg2d, delta,
                  u_id_o, v_id_o,
                  idx_u, idx_v, idx_n, r0, r1, r2, r3, potc, dbuf,
                  gs0, gs1, gs2, gs3, ws0, ws1, ws2, ws3, dsem):
    wid = lax.axis_index("s") * 2 + lax.axis_index("c")
    base = wid * _RPW

    ring = (r0, r1, r2, r3)
    gsem = (gs0, gs1, gs2, gs3)
    wsem = (ws0, ws1, ws2, ws3)

    pltpu.sync_copy(nu2d.at[pl.ds(4 * wid, 4)], idx_u)
    pltpu.sync_copy(nv2d.at[pl.ds(4 * wid, 4)], idx_v)
    pltpu.sync_copy(neg2d.at[pl.ds(16 * wid, 16)], idx_n)

    jobs = []
    for c in range(4):
        for m in range(4):
            jobs.append(("neg", v_emb, idx_n, 4 * c + m, None, 32 * m))
        jobs.append(("plain", u_emb, idx_u, c, u_id_o, base + _CH * c))
        jobs.append(("vblend", v_emb, idx_v, c, v_id_o, base + _CH * c))

    g = [None, None, None, None]
    w = [None, None, None, None]
    dcp = [None]

    def pool(src, off):
        @plsc.parallel_loop(0, 32, unroll=2)
        def _row(r):
            for j in range(_D // 16):
                sl = pl.ds(16 * j, 16)
                s = ((src[4 * r, sl] + src[4 * r + 1, sl])
                     + (src[4 * r + 2, sl] + src[4 * r + 3, sl]))
                potc[off + r, sl] = s * 0.25

    def blend(src):
        @plsc.parallel_loop(0, _CH, unroll=4)
        def _row(r):
            for j in range(_D // 16):
                sl = pl.ds(16 * j, 16)
                p = potc[r, sl]
                src[r, sl] = p + dbuf[r, sl] * (src[r, sl] - p)

    for i in range(len(jobs) + 3):
        if i < len(jobs):
            s = i % 4
            if i >= 4 and w[s] is not None:
                w[s].wait()
                w[s] = None
            kind, tbl, idx, row, out, off = jobs[i]
            g[s] = pltpu.async_copy(tbl.at[idx.at[row]], ring[s], gsem[s])
            if kind == "vblend":
                dcp[0] = pltpu.async_copy(delta.at[pl.ds(off, _CH)], dbuf, dsem)
        p = i - 3
        if 0 <= p < len(jobs):
            sp = p % 4
            g[sp].wait()
            kind, tbl, idx, row, out, off = jobs[p]
            if kind == "neg":
                pool(ring[sp], off)
                w[sp] = None
            elif kind == "vblend":
                dcp[0].wait()
                blend(ring[sp])
                w[sp] = pltpu.async_copy(ring[sp], out.at[pl.ds(off, _CH)],
                                         wsem[sp])
            else:
                w[sp] = pltpu.async_copy(ring[sp], out.at[pl.ds(off, _CH)],
                                         wsem[sp])
    for s in range(4):
        if w[s] is not None:
            w[s].wait()


def _sc_rev_body(u_rev, v_rev, nu2d, nv2d, u_rev_o, v_rev_o,
                 idx_u, idx_v, r0, r1, r2, r3,
                 gs0, gs1, gs2, gs3, ws0, ws1, ws2, ws3):
    wid = lax.axis_index("s") * 2 + lax.axis_index("c")
    base = wid * _RPW

    ring = (r0, r1, r2, r3)
    gsem = (gs0, gs1, gs2, gs3)
    wsem = (ws0, ws1, ws2, ws3)

    pltpu.sync_copy(nu2d.at[pl.ds(4 * wid, 4)], idx_u)
    pltpu.sync_copy(nv2d.at[pl.ds(4 * wid, 4)], idx_v)

    jobs = []
    for c in range(4):
        jobs.append((u_rev, idx_u, c, u_rev_o, base + _CH * c))
        jobs.append((v_rev, idx_v, c, v_rev_o, base + _CH * c))

    g = [None, None, None, None]
    w = [None, None, None, None]
    for i in range(len(jobs) + 3):
        if i < len(jobs):
            s = i % 4
            if w[s] is not None:
                w[s].wait()
                w[s] = None
            tbl, idx, row, out, off = jobs[i]
            g[s] = pltpu.async_copy(tbl.at[idx.at[row]], ring[s], gsem[s])
        p = i - 3
        if 0 <= p < len(jobs):
            sp = p % 4
            g[sp].wait()
            tbl, idx, row, out, off = jobs[p]
            w[sp] = pltpu.async_copy(ring[sp], out.at[pl.ds(off, _CH)],
                                     wsem[sp])
    for s in range(4):
        if w[s] is not None:
            w[s].wait()


def _sc_gather_main(u_emb, v_emb, nu2d, nv2d, neg2d, delta):
    f32 = jnp.float32
    out = jax.ShapeDtypeStruct((_B, _D), f32)
    run = pl.kernel(
        _sc_main_body,
        mesh=plsc.VectorSubcoreMesh(core_axis_name="c", subcore_axis_name="s"),
        out_type=[out, out],
        scratch_types=(
            [pltpu.VMEM((4, 128), jnp.int32),
             pltpu.VMEM((4, 128), jnp.int32),
             pltpu.VMEM((16, 128), jnp.int32)]
            + [pltpu.VMEM((_CH, _D), f32) for _ in range(4)]
            + [pltpu.VMEM((_CH, _D), f32),
               pltpu.VMEM((_CH, _D), f32)]
            + [pltpu.SemaphoreType.DMA for _ in range(9)]
        ),
    )
    return run(u_emb, v_emb, nu2d, nv2d, neg2d, delta)


def _sc_gather_rev(u_rev, v_rev, nu2d, nv2d):
    f32 = jnp.float32
    out = jax.ShapeDtypeStruct((_B, _D), f32)
    run = pl.kernel(
        _sc_rev_body,
        mesh=plsc.VectorSubcoreMesh(core_axis_name="c", subcore_axis_name="s"),
        out_type=[out, out],
        scratch_types=(
            [pltpu.VMEM((4, 128), jnp.int32),
             pltpu.VMEM((4, 128), jnp.int32)]
            + [pltpu.VMEM((_CH, _D), f32) for _ in range(4)]
            + [pltpu.SemaphoreType.DMA for _ in range(8)]
        ),
    )
    return run(u_rev, v_rev, nu2d, nv2d)


def _blend_body(vraw, pot, delta, out):
    p = pot[:]
    out[:] = p + delta[:] * (vraw[:] - p)


def _blend(vraw, pot, delta):
    blk = pl.BlockSpec((1024, _D), lambda i: (i, 0))
    return pl.pallas_call(
        _blend_body,
        grid=(_B // 1024,),
        in_specs=[blk, blk, blk],
        out_specs=blk,
        out_shape=jax.ShapeDtypeStruct((_B, _D), jnp.float32),
    )(vraw, pot, delta)


_MBLK = 4096
_NBLK = _B // _MBLK


def _mlp_body(u_ref, v_ref, w1u, w1v, b1, g1, be1, w2, b2, g2, be2,
              w3, b3, g3, be3, wc, bc, out_ref,
              z1s, z2s, z3s, s1, q1, s2, q2, s3, q3):
    eps = 1e-5
    inv_b = 1.0 / _B
    ph = pl.program_id(0)
    b = pl.program_id(1)

    def mm(x, w):
        return jnp.dot(x, w, preferred_element_type=jnp.float32)

    def acc(sref, qref, z):
        @pl.when(b == 0)
        def _():
            sref[:] = jnp.zeros_like(sref)
            qref[:] = jnp.zeros_like(qref)
        sref[:] += jnp.sum(z, axis=0, keepdims=True)
        qref[:] += jnp.sum(z * z, axis=0, keepdims=True)

    def norm(sref, qref, z, g, be):
        mu = sref[:] * inv_b
        var = qref[:] * inv_b - mu * mu
        return g[:] * (z - mu) * lax.rsqrt(var + eps) + be[:]

    @pl.when(ph == 0)
    def _():
        z1 = jnp.maximum(mm(u_ref[:], w1u[:]) + mm(v_ref[:], w1v[:]) + b1[:],
                         0.0)
        z1s[pl.ds(b * _MBLK, _MBLK), :] = z1
        acc(s1, q1, z1)

    @pl.when(ph == 1)
    def _():
        h1 = norm(s1, q1, z1s[pl.ds(b * _MBLK, _MBLK), :], g1, be1)
        z2 = jnp.maximum(mm(h1, w2[:]) + b2[:], 0.0)
        z2s[pl.ds(b * _MBLK, _MBLK), :] = z2
        acc(s2, q2, z2)

    @pl.when(ph == 2)
    def _():
        h2 = norm(s2, q2, z2s[pl.ds(b * _MBLK, _MBLK), :], g2, be2)
        z3 = jnp.maximum(mm(h2, w3[:]) + b3[:], 0.0)
        z3s[pl.ds(b * _MBLK, _MBLK), :] = z3
        acc(s3, q3, z3)

    @pl.when(ph == 3)
    def _():
        h3 = norm(s3, q3, z3s[pl.ds(b * _MBLK, _MBLK), :], g3, be3)
        logit = mm(h3, wc[:]) + bc[:]
        out_ref[:] = 1.0 / (1.0 + jnp.exp(-logit))


def _mlp(u_id, v_id, W1, b1, W2, b2, W3, b3, Wc, bc, g1, be1, g2, be2, g3, be3):
    f32 = jnp.float32
    io_spec = pl.BlockSpec(
        (_MBLK, _D), lambda ph, b: (jnp.where(ph == 0, b, 0), 0))
    full = lambda r, c: pl.BlockSpec((r, c), lambda ph, b: (0, 0))
    return pl.pallas_call(
        _mlp_body,
        grid=(4, _NBLK),
        in_specs=[io_spec, io_spec,
                  full(_D, _D), full(_D, _D), full(1, _D), full(1, _D),
                  full(1, _D), full(_D, _D // 2), full(1, _D // 2),
                  full(1, _D // 2), full(1, _D // 2), full(_D // 2, _D // 4),
                  full(1, _D // 4), full(1, _D // 4), full(1, _D // 4),
                  full(_D // 4, 1), full(1, 1)],
        out_specs=pl.BlockSpec((_MBLK, 1),
                               lambda ph, b: (jnp.where(ph == 3, b, 0), 0)),
        out_shape=jax.ShapeDtypeStruct((_B, 1), f32),
        scratch_shapes=[
            pltpu.VMEM((_B, _D), f32),
            pltpu.VMEM((_B, _D // 2), f32),
            pltpu.VMEM((_B, _D // 4), f32),
            pltpu.VMEM((1, _D), f32), pltpu.VMEM((1, _D), f32),
            pltpu.VMEM((1, _D // 2), f32), pltpu.VMEM((1, _D // 2), f32),
            pltpu.VMEM((1, _D // 4), f32), pltpu.VMEM((1, _D // 4), f32),
        ],
    )(u_id, v_id,
      W1[:, :_D].T, W1[:, _D:].T, b1.reshape(1, -1), g1.reshape(1, -1),
      be1.reshape(1, -1), W2.T, b2.reshape(1, -1), g2.reshape(1, -1),
      be2.reshape(1, -1), W3.T, b3.reshape(1, -1), g3.reshape(1, -1),
      be3.reshape(1, -1), Wc.T, bc.reshape(1, -1))


def kernel(nodes_u, nodes_v, global_protos, inter_nums, u_emb_w, v_emb_w,
           u_rev_w, v_rev_w, W1, b1, W2, b2, W3, b3, Wc, bc,
           g1, be1, g2, be2, g3, be3):
    neg2d, delta = _rng_consts()
    nu2d = nodes_u.astype(jnp.int32).reshape(_B // 128, 128)
    nv2d = nodes_v.astype(jnp.int32).reshape(_B // 128, 128)
    u_id, v_id = _sc_gather_main(u_emb_w, v_emb_w, nu2d, nv2d, neg2d, delta)
    u_review, v_review = _sc_gather_rev(u_rev_w, v_rev_w, nu2d, nv2d)
    pred = _mlp(u_id, v_id, W1, b1, W2, b2, W3, b3, Wc, bc,
                g1, be1, g2, be2, g3, be3)
    return (pred[:, 0], u_id, v_id, u_review, v_review)


# speedup vs baseline: 1.2587x; 1.0299x over previous
import jax
import jax.numpy as jnp
from jax import lax
from jax.experimental import pallas as pl
from jax.experimental.pallas import tpu as pltpu
from jax.experimental.pallas import tpu_sc as plsc

_B = 16384
_D = 128
_ITEM_NUM = 100000
_NW = 32
_RPW = _B // _NW
_CH = 128


def _rng_consts():
    with jax.ensure_compile_time_eval():
        kk = jax.random.key(42)
        k1, k2 = jax.random.split(kk)
        neg = jax.random.randint(k1, (_B, 4), 0, _ITEM_NUM)
        delta = jnp.clip(
            jax.random.normal(k2, (_B, _D), jnp.float32) * 0.1 + 0.5, 0.0, 1.0)
        neg2d = neg.reshape(_B * 4 // 128, 128).astype(jnp.int32)
    return neg2d, delta


_NR = 5
_LAG = _NR - 1


def _sc_main_body(u_emb, v_emb, nu2d, nv2d, neg2d, delta,
                  u_id_o, v_id_o, *scr):
    wid = lax.axis_index("s") * 2 + lax.axis_index("c")
    base = wid * _RPW

    idx_u, idx_v, idx_n = scr[0:3]
    ring = scr[3:3 + _NR]
    potc, dbuf = scr[3 + _NR:5 + _NR]
    gsem = scr[5 + _NR:5 + 2 * _NR]
    wsem = scr[5 + 2 * _NR:5 + 3 * _NR]
    dsem = scr[5 + 3 * _NR]

    pltpu.sync_copy(nu2d.at[pl.ds(4 * wid, 4)], idx_u)
    pltpu.sync_copy(nv2d.at[pl.ds(4 * wid, 4)], idx_v)
    pltpu.sync_copy(neg2d.at[pl.ds(16 * wid, 16)], idx_n)

    jobs = []
    for c in range(4):
        for m in range(4):
            jobs.append(("neg", v_emb, idx_n, 4 * c + m, None, 32 * m))
        jobs.append(("plain", u_emb, idx_u, c, u_id_o, base + _CH * c))
        jobs.append(("vblend", v_emb, idx_v, c, v_id_o, base + _CH * c))

    g = [None] * _NR
    w = [None] * _NR
    dcp = [None]

    def pool(src, off):
        @plsc.parallel_loop(0, 32, unroll=2)
        def _row(r):
            for j in range(_D // 16):
                sl = pl.ds(16 * j, 16)
                s = ((src[4 * r, sl] + src[4 * r + 1, sl])
                     + (src[4 * r + 2, sl] + src[4 * r + 3, sl]))
                potc[off + r, sl] = s * 0.25

    def blend(src):
        @plsc.parallel_loop(0, _CH, unroll=4)
        def _row(r):
            for j in range(_D // 16):
                sl = pl.ds(16 * j, 16)
                p = potc[r, sl]
                src[r, sl] = p + dbuf[r, sl] * (src[r, sl] - p)

    for i in range(len(jobs) + _LAG):
        if i < len(jobs):
            s = i % _NR
            if i >= _NR and w[s] is not None:
                w[s].wait()
                w[s] = None
            kind, tbl, idx, row, out, off = jobs[i]
            g[s] = pltpu.async_copy(tbl.at[idx.at[row]], ring[s], gsem[s])
            if kind == "vblend":
                dcp[0] = pltpu.async_copy(delta.at[pl.ds(off, _CH)], dbuf, dsem)
        p = i - _LAG
        if 0 <= p < len(jobs):
            sp = p % _NR
            g[sp].wait()
            kind, tbl, idx, row, out, off = jobs[p]
            if kind == "neg":
                pool(ring[sp], off)
                w[sp] = None
            elif kind == "vblend":
                dcp[0].wait()
                blend(ring[sp])
                w[sp] = pltpu.async_copy(ring[sp], out.at[pl.ds(off, _CH)],
                                         wsem[sp])
            else:
                w[sp] = pltpu.async_copy(ring[sp], out.at[pl.ds(off, _CH)],
                                         wsem[sp])
    for s in range(_NR):
        if w[s] is not None:
            w[s].wait()


def _sc_rev_body(u_rev, v_rev, nu2d, nv2d, u_rev_o, v_rev_o,
                 idx_u, idx_v, r0, r1, r2, r3,
                 gs0, gs1, gs2, gs3, ws0, ws1, ws2, ws3):
    wid = lax.axis_index("s") * 2 + lax.axis_index("c")
    base = wid * _RPW

    ring = (r0, r1, r2, r3)
    gsem = (gs0, gs1, gs2, gs3)
    wsem = (ws0, ws1, ws2, ws3)

    pltpu.sync_copy(nu2d.at[pl.ds(4 * wid, 4)], idx_u)
    pltpu.sync_copy(nv2d.at[pl.ds(4 * wid, 4)], idx_v)

    jobs = []
    for c in range(4):
        jobs.append((u_rev, idx_u, c, u_rev_o, base + _CH * c))
        jobs.append((v_rev, idx_v, c, v_rev_o, base + _CH * c))

    g = [None, None, None, None]
    w = [None, None, None, None]
    for i in range(len(jobs) + 3):
        if i < len(jobs):
            s = i % 4
            if w[s] is not None:
                w[s].wait()
                w[s] = None
            tbl, idx, row, out, off = jobs[i]
            g[s] = pltpu.async_copy(tbl.at[idx.at[row]], ring[s], gsem[s])
        p = i - 3
        if 0 <= p < len(jobs):
            sp = p % 4
            g[sp].wait()
            tbl, idx, row, out, off = jobs[p]
            w[sp] = pltpu.async_copy(ring[sp], out.at[pl.ds(off, _CH)],
                                     wsem[sp])
    for s in range(4):
        if w[s] is not None:
            w[s].wait()


def _sc_gather_main(u_emb, v_emb, nu2d, nv2d, neg2d, delta):
    f32 = jnp.float32
    out = jax.ShapeDtypeStruct((_B, _D), f32)
    run = pl.kernel(
        _sc_main_body,
        mesh=plsc.VectorSubcoreMesh(core_axis_name="c", subcore_axis_name="s"),
        out_type=[out, out],
        scratch_types=(
            [pltpu.VMEM((4, 128), jnp.int32),
             pltpu.VMEM((4, 128), jnp.int32),
             pltpu.VMEM((16, 128), jnp.int32)]
            + [pltpu.VMEM((_CH, _D), f32) for _ in range(_NR)]
            + [pltpu.VMEM((_CH, _D), f32),
               pltpu.VMEM((_CH, _D), f32)]
            + [pltpu.SemaphoreType.DMA for _ in range(2 * _NR + 1)]
        ),
    )
    return run(u_emb, v_emb, nu2d, nv2d, neg2d, delta)


def _sc_gather_rev(u_rev, v_rev, nu2d, nv2d):
    f32 = jnp.float32
    out = jax.ShapeDtypeStruct((_B, _D), f32)
    run = pl.kernel(
        _sc_rev_body,
        mesh=plsc.VectorSubcoreMesh(core_axis_name="c", subcore_axis_name="s"),
        out_type=[out, out],
        scratch_types=(
            [pltpu.VMEM((4, 128), jnp.int32),
             pltpu.VMEM((4, 128), jnp.int32)]
            + [pltpu.VMEM((_CH, _D), f32) for _ in range(4)]
            + [pltpu.SemaphoreType.DMA for _ in range(8)]
        ),
    )
    return run(u_rev, v_rev, nu2d, nv2d)


def _blend_body(vraw, pot, delta, out):
    p = pot[:]
    out[:] = p + delta[:] * (vraw[:] - p)


def _blend(vraw, pot, delta):
    blk = pl.BlockSpec((1024, _D), lambda i: (i, 0))
    return pl.pallas_call(
        _blend_body,
        grid=(_B // 1024,),
        in_specs=[blk, blk, blk],
        out_specs=blk,
        out_shape=jax.ShapeDtypeStruct((_B, _D), jnp.float32),
    )(vraw, pot, delta)


_MBLK = 4096
_NBLK = _B // _MBLK


def _mlp_body(u_ref, v_ref, w1u, w1v, b1, g1, be1, w2, b2, g2, be2,
              w3, b3, g3, be3, wc, bc, out_ref,
              z1s, z2s, z3s, s1, q1, s2, q2, s3, q3):
    eps = 1e-5
    inv_b = 1.0 / _B
    ph = pl.program_id(0)
    b = pl.program_id(1)

    def mm(x, w):
        return jnp.dot(x, w, preferred_element_type=jnp.float32)

    def acc(sref, qref, z):
        @pl.when(b == 0)
        def _():
            sref[:] = jnp.zeros_like(sref)
            qref[:] = jnp.zeros_like(qref)
        sref[:] += jnp.sum(z, axis=0, keepdims=True)
        qref[:] += jnp.sum(z * z, axis=0, keepdims=True)

    def norm(sref, qref, z, g, be):
        mu = sref[:] * inv_b
        var = qref[:] * inv_b - mu * mu
        return g[:] * (z - mu) * lax.rsqrt(var + eps) + be[:]

    @pl.when(ph == 0)
    def _():
        z1 = jnp.maximum(mm(u_ref[:], w1u[:]) + mm(v_ref[:], w1v[:]) + b1[:],
                         0.0)
        z1s[pl.ds(b * _MBLK, _MBLK), :] = z1
        acc(s1, q1, z1)

    @pl.when(ph == 1)
    def _():
        h1 = norm(s1, q1, z1s[pl.ds(b * _MBLK, _MBLK), :], g1, be1)
        z2 = jnp.maximum(mm(h1, w2[:]) + b2[:], 0.0)
        z2s[pl.ds(b * _MBLK, _MBLK), :] = z2
        acc(s2, q2, z2)

    @pl.when(ph == 2)
    def _():
        h2 = norm(s2, q2, z2s[pl.ds(b * _MBLK, _MBLK), :], g2, be2)
        z3 = jnp.maximum(mm(h2, w3[:]) + b3[:], 0.0)
        z3s[pl.ds(b * _MBLK, _MBLK), :] = z3
        acc(s3, q3, z3)

    @pl.when(ph == 3)
    def _():
        h3 = norm(s3, q3, z3s[pl.ds(b * _MBLK, _MBLK), :], g3, be3)
        logit = mm(h3, wc[:]) + bc[:]
        out_ref[:] = 1.0 / (1.0 + jnp.exp(-logit))


def _mlp(u_id, v_id, W1, b1, W2, b2, W3, b3, Wc, bc, g1, be1, g2, be2, g3, be3):
    f32 = jnp.float32
    io_spec = pl.BlockSpec(
        (_MBLK, _D), lambda ph, b: (jnp.where(ph == 0, b, 0), 0))
    full = lambda r, c: pl.BlockSpec((r, c), lambda ph, b: (0, 0))
    return pl.pallas_call(
        _mlp_body,
        grid=(4, _NBLK),
        in_specs=[io_spec, io_spec,
                  full(_D, _D), full(_D, _D), full(1, _D), full(1, _D),
                  full(1, _D), full(_D, _D // 2), full(1, _D // 2),
                  full(1, _D // 2), full(1, _D // 2), full(_D // 2, _D // 4),
                  full(1, _D // 4), full(1, _D // 4), full(1, _D // 4),
                  full(_D // 4, 1), full(1, 1)],
        out_specs=pl.BlockSpec((_MBLK, 1),
                               lambda ph, b: (jnp.where(ph == 3, b, 0), 0)),
        out_shape=jax.ShapeDtypeStruct((_B, 1), f32),
        scratch_shapes=[
            pltpu.VMEM((_B, _D), f32),
            pltpu.VMEM((_B, _D // 2), f32),
            pltpu.VMEM((_B, _D // 4), f32),
            pltpu.VMEM((1, _D), f32), pltpu.VMEM((1, _D), f32),
            pltpu.VMEM((1, _D // 2), f32), pltpu.VMEM((1, _D // 2), f32),
            pltpu.VMEM((1, _D // 4), f32), pltpu.VMEM((1, _D // 4), f32),
        ],
    )(u_id, v_id,
      W1[:, :_D].T, W1[:, _D:].T, b1.reshape(1, -1), g1.reshape(1, -1),
      be1.reshape(1, -1), W2.T, b2.reshape(1, -1), g2.reshape(1, -1),
      be2.reshape(1, -1), W3.T, b3.reshape(1, -1), g3.reshape(1, -1),
      be3.reshape(1, -1), Wc.T, bc.reshape(1, -1))


def kernel(nodes_u, nodes_v, global_protos, inter_nums, u_emb_w, v_emb_w,
           u_rev_w, v_rev_w, W1, b1, W2, b2, W3, b3, Wc, bc,
           g1, be1, g2, be2, g3, be3):
    neg2d, delta = _rng_consts()
    nu2d = nodes_u.astype(jnp.int32).reshape(_B // 128, 128)
    nv2d = nodes_v.astype(jnp.int32).reshape(_B // 128, 128)
    u_id, v_id = _sc_gather_main(u_emb_w, v_emb_w, nu2d, nv2d, neg2d, delta)
    u_review, v_review = _sc_gather_rev(u_rev_w, v_rev_w, nu2d, nv2d)
    pred = _mlp(u_id, v_id, W1, b1, W2, b2, W3, b3, Wc, bc,
                g1, be1, g2, be2, g3, be3)
    return (pred[:, 0], u_id, v_id, u_review, v_review)


# speedup vs baseline: 1.2869x; 1.0224x over previous
import jax
import jax.numpy as jnp
from jax import lax
from jax.experimental import pallas as pl
from jax.experimental.pallas import tpu as pltpu
from jax.experimental.pallas import tpu_sc as plsc

_B = 16384
_D = 128
_ITEM_NUM = 100000
_NW = 32
_RPW = _B // _NW
_CH = 128


def _rng_consts():
    with jax.ensure_compile_time_eval():
        kk = jax.random.key(42)
        k1, k2 = jax.random.split(kk)
        neg = jax.random.randint(k1, (_B, 4), 0, _ITEM_NUM)
        delta = jnp.clip(
            jax.random.normal(k2, (_B, _D), jnp.float32) * 0.1 + 0.5, 0.0, 1.0)
        neg2d = neg.reshape(_B * 4 // 128, 128).astype(jnp.int32)
    return neg2d, delta


_NR = 5
_LAG = _NR - 1


def _sc_main_body(u_emb, v_emb, nu2d, nv2d, neg2d, delta,
                  u_id_o, v_id_o, *scr):
    wid = lax.axis_index("s") * 2 + lax.axis_index("c")
    base = wid * _RPW

    idx_u, idx_v, idx_n = scr[0:3]
    ring = scr[3:3 + _NR]
    potc, dbuf = scr[3 + _NR:5 + _NR]
    gsem = scr[5 + _NR:5 + 2 * _NR]
    wsem = scr[5 + 2 * _NR:5 + 3 * _NR]
    dsem = scr[5 + 3 * _NR]

    pltpu.sync_copy(nu2d.at[pl.ds(4 * wid, 4)], idx_u)
    pltpu.sync_copy(nv2d.at[pl.ds(4 * wid, 4)], idx_v)
    pltpu.sync_copy(neg2d.at[pl.ds(16 * wid, 16)], idx_n)

    jobs = []
    for c in range(4):
        for m in range(4):
            jobs.append(("neg", v_emb, idx_n, 4 * c + m, None, 32 * m))
        jobs.append(("plain", u_emb, idx_u, c, u_id_o, base + _CH * c))
        jobs.append(("vblend", v_emb, idx_v, c, v_id_o, base + _CH * c))

    g = [None] * _NR
    w = [None] * _NR
    dcp = [None]

    def pool(src, off):
        @plsc.parallel_loop(0, 32, unroll=2)
        def _row(r):
            for j in range(_D // 16):
                sl = pl.ds(16 * j, 16)
                s = ((src[4 * r, sl] + src[4 * r + 1, sl])
                     + (src[4 * r + 2, sl] + src[4 * r + 3, sl]))
                potc[off + r, sl] = s * 0.25

    def blend(src):
        @plsc.parallel_loop(0, _CH, unroll=4)
        def _row(r):
            for j in range(_D // 16):
                sl = pl.ds(16 * j, 16)
                p = potc[r, sl]
                src[r, sl] = p + dbuf[r, sl] * (src[r, sl] - p)

    for i in range(len(jobs) + _LAG):
        if i < len(jobs):
            s = i % _NR
            if i >= _NR and w[s] is not None:
                w[s].wait()
                w[s] = None
            kind, tbl, idx, row, out, off = jobs[i]
            g[s] = pltpu.async_copy(tbl.at[idx.at[row]], ring[s], gsem[s])
            if kind == "vblend":
                dcp[0] = pltpu.async_copy(delta.at[pl.ds(off, _CH)], dbuf, dsem)
        p = i - _LAG
        if 0 <= p < len(jobs):
            sp = p % _NR
            g[sp].wait()
            kind, tbl, idx, row, out, off = jobs[p]
            if kind == "neg":
                pool(ring[sp], off)
                w[sp] = None
            elif kind == "vblend":
                dcp[0].wait()
                blend(ring[sp])
                w[sp] = pltpu.async_copy(ring[sp], out.at[pl.ds(off, _CH)],
                                         wsem[sp])
            else:
                w[sp] = pltpu.async_copy(ring[sp], out.at[pl.ds(off, _CH)],
                                         wsem[sp])
    for s in range(_NR):
        if w[s] is not None:
            w[s].wait()


def _sc_rev_body(u_rev, v_rev, nu2d, nv2d, u_rev_o, v_rev_o,
                 idx_u, idx_v, r0, r1, r2, r3,
                 gs0, gs1, gs2, gs3, ws0, ws1, ws2, ws3):
    wid = lax.axis_index("s") * 2 + lax.axis_index("c")
    base = wid * _RPW

    ring = (r0, r1, r2, r3)
    gsem = (gs0, gs1, gs2, gs3)
    wsem = (ws0, ws1, ws2, ws3)

    pltpu.sync_copy(nu2d.at[pl.ds(4 * wid, 4)], idx_u)
    pltpu.sync_copy(nv2d.at[pl.ds(4 * wid, 4)], idx_v)

    jobs = []
    for c in range(4):
        jobs.append((u_rev, idx_u, c, u_rev_o, base + _CH * c))
        jobs.append((v_rev, idx_v, c, v_rev_o, base + _CH * c))

    g = [None, None, None, None]
    w = [None, None, None, None]
    for i in range(len(jobs) + 3):
        if i < len(jobs):
            s = i % 4
            if w[s] is not None:
                w[s].wait()
                w[s] = None
            tbl, idx, row, out, off = jobs[i]
            g[s] = pltpu.async_copy(tbl.at[idx.at[row]], ring[s], gsem[s])
        p = i - 3
        if 0 <= p < len(jobs):
            sp = p % 4
            g[sp].wait()
            tbl, idx, row, out, off = jobs[p]
            w[sp] = pltpu.async_copy(ring[sp], out.at[pl.ds(off, _CH)],
                                     wsem[sp])
    for s in range(4):
        if w[s] is not None:
            w[s].wait()


def _sc_gather_main(u_emb, v_emb, nu2d, nv2d, neg2d, delta):
    f32 = jnp.float32
    out = jax.ShapeDtypeStruct((_B, _D), f32)
    run = pl.kernel(
        _sc_main_body,
        mesh=plsc.VectorSubcoreMesh(core_axis_name="c", subcore_axis_name="s"),
        out_type=[out, out],
        scratch_types=(
            [pltpu.VMEM((4, 128), jnp.int32),
             pltpu.VMEM((4, 128), jnp.int32),
             pltpu.VMEM((16, 128), jnp.int32)]
            + [pltpu.VMEM((_CH, _D), f32) for _ in range(_NR)]
            + [pltpu.VMEM((_CH, _D), f32),
               pltpu.VMEM((_CH, _D), f32)]
            + [pltpu.SemaphoreType.DMA for _ in range(2 * _NR + 1)]
        ),
    )
    return run(u_emb, v_emb, nu2d, nv2d, neg2d, delta)


def _sc_gather_rev(u_rev, v_rev, nu2d, nv2d):
    f32 = jnp.float32
    out = jax.ShapeDtypeStruct((_B, _D), f32)
    run = pl.kernel(
        _sc_rev_body,
        mesh=plsc.VectorSubcoreMesh(core_axis_name="c", subcore_axis_name="s"),
        out_type=[out, out],
        scratch_types=(
            [pltpu.VMEM((4, 128), jnp.int32),
             pltpu.VMEM((4, 128), jnp.int32)]
            + [pltpu.VMEM((_CH, _D), f32) for _ in range(4)]
            + [pltpu.SemaphoreType.DMA for _ in range(8)]
        ),
    )
    return run(u_rev, v_rev, nu2d, nv2d)


def _blend_body(vraw, pot, delta, out):
    p = pot[:]
    out[:] = p + delta[:] * (vraw[:] - p)


def _blend(vraw, pot, delta):
    blk = pl.BlockSpec((1024, _D), lambda i: (i, 0))
    return pl.pallas_call(
        _blend_body,
        grid=(_B // 1024,),
        in_specs=[blk, blk, blk],
        out_specs=blk,
        out_shape=jax.ShapeDtypeStruct((_B, _D), jnp.float32),
    )(vraw, pot, delta)


_MBLK = 4096
_NBLK = _B // _MBLK


def _mlp_body(u_ref, v_ref, w1u, w1v, b1, g1, be1, w2, b2, g2, be2,
              w3, b3, g3, be3, wc, bc, out_ref,
              z1s, z2s, z3s, s1, q1, s2, q2, s3, q3):
    eps = 1e-5
    inv_b = 1.0 / _B
    ph = pl.program_id(0)
    b = pl.program_id(1)

    def mm(x, w):
        return jnp.dot(x, w, preferred_element_type=jnp.float32)

    def acc(sref, qref, z):
        @pl.when(b == 0)
        def _():
            sref[:] = jnp.zeros_like(sref)
            qref[:] = jnp.zeros_like(qref)
        sref[:] += jnp.sum(z, axis=0, keepdims=True)
        qref[:] += jnp.sum(z * z, axis=0, keepdims=True)

    def norm(sref, qref, z, g, be):
        mu = sref[:] * inv_b
        var = qref[:] * inv_b - mu * mu
        return g[:] * (z - mu) * lax.rsqrt(var + eps) + be[:]

    @pl.when(ph == 0)
    def _():
        z1 = jnp.maximum(mm(u_ref[:], w1u[:]) + mm(v_ref[:], w1v[:]) + b1[:],
                         0.0)
        z1s[pl.ds(b * _MBLK, _MBLK), :] = z1
        acc(s1, q1, z1)

    @pl.when(ph == 1)
    def _():
        h1 = norm(s1, q1, z1s[pl.ds(b * _MBLK, _MBLK), :], g1, be1)
        z2 = jnp.maximum(mm(h1, w2[:]) + b2[:], 0.0)
        z2s[pl.ds(b * _MBLK, _MBLK), :] = z2
        acc(s2, q2, z2)

    @pl.when(ph == 2)
    def _():
        h2 = norm(s2, q2, z2s[pl.ds(b * _MBLK, _MBLK), :], g2, be2)
        z3 = jnp.maximum(mm(h2, w3[:]) + b3[:], 0.0)
        z3s[pl.ds(b * _MBLK, _MBLK), :] = z3
        acc(s3, q3, z3)

    @pl.when(ph == 3)
    def _():
        h3 = norm(s3, q3, z3s[pl.ds(b * _MBLK, _MBLK), :], g3, be3)
        logit = mm(h3, wc[:]) + bc[:]
        out_ref[:] = 1.0 / (1.0 + jnp.exp(-logit))


def _mlp(u_id, v_id, W1, b1, W2, b2, W3, b3, Wc, bc, g1, be1, g2, be2, g3, be3):
    f32 = jnp.float32
    io_spec = pl.BlockSpec(
        (_MBLK, _D), lambda ph, b: (jnp.where(ph == 0, b, _NBLK - 1), 0))
    full = lambda r, c: pl.BlockSpec((r, c), lambda ph, b: (0, 0))
    return pl.pallas_call(
        _mlp_body,
        grid=(4, _NBLK),
        in_specs=[io_spec, io_spec,
                  full(_D, _D), full(_D, _D), full(1, _D), full(1, _D),
                  full(1, _D), full(_D, _D // 2), full(1, _D // 2),
                  full(1, _D // 2), full(1, _D // 2), full(_D // 2, _D // 4),
                  full(1, _D // 4), full(1, _D // 4), full(1, _D // 4),
                  full(_D // 4, 1), full(1, 1)],
        out_specs=pl.BlockSpec((_MBLK, 1),
                               lambda ph, b: (jnp.where(ph == 3, b, 0), 0)),
        out_shape=jax.ShapeDtypeStruct((_B, 1), f32),
        scratch_shapes=[
            pltpu.VMEM((_B, _D), f32),
            pltpu.VMEM((_B, _D // 2), f32),
            pltpu.VMEM((_B, _D // 4), f32),
            pltpu.VMEM((1, _D), f32), pltpu.VMEM((1, _D), f32),
            pltpu.VMEM((1, _D // 2), f32), pltpu.VMEM((1, _D // 2), f32),
            pltpu.VMEM((1, _D // 4), f32), pltpu.VMEM((1, _D // 4), f32),
        ],
    )(u_id, v_id,
      W1[:, :_D].T, W1[:, _D:].T, b1.reshape(1, -1), g1.reshape(1, -1),
      be1.reshape(1, -1), W2.T, b2.reshape(1, -1), g2.reshape(1, -1),
      be2.reshape(1, -1), W3.T, b3.reshape(1, -1), g3.reshape(1, -1),
      be3.reshape(1, -1), Wc.T, bc.reshape(1, -1))


def kernel(nodes_u, nodes_v, global_protos, inter_nums, u_emb_w, v_emb_w,
           u_rev_w, v_rev_w, W1, b1, W2, b2, W3, b3, Wc, bc,
           g1, be1, g2, be2, g3, be3):
    neg2d, delta = _rng_consts()
    nu2d = nodes_u.astype(jnp.int32).reshape(_B // 128, 128)
    nv2d = nodes_v.astype(jnp.int32).reshape(_B // 128, 128)
    u_id, v_id = _sc_gather_main(u_emb_w, v_emb_w, nu2d, nv2d, neg2d, delta)
    u_review, v_review = _sc_gather_rev(u_rev_w, v_rev_w, nu2d, nv2d)
    pred = _mlp(u_id, v_id, W1, b1, W2, b2, W3, b3, Wc, bc,
                g1, be1, g2, be2, g3, be3)
    return (pred[:, 0], u_id, v_id, u_review, v_review)


# speedup vs baseline: 1.3012x; 1.0111x over previous
import jax
import jax.numpy as jnp
from jax import lax
from jax.experimental import pallas as pl
from jax.experimental.pallas import tpu as pltpu
from jax.experimental.pallas import tpu_sc as plsc

_B = 16384
_D = 128
_ITEM_NUM = 100000
_NW = 32
_RPW = _B // _NW
_CH = 128


def _rng_consts():
    with jax.ensure_compile_time_eval():
        kk = jax.random.key(42)
        k1, k2 = jax.random.split(kk)
        neg = jax.random.randint(k1, (_B, 4), 0, _ITEM_NUM)
        delta = jnp.clip(
            jax.random.normal(k2, (_B, _D), jnp.float32) * 0.1 + 0.5, 0.0, 1.0)
        neg2d = neg.reshape(_B * 4 // 128, 128).astype(jnp.int32)
    return neg2d, delta


_NR = 5
_LAG = _NR - 1


def _sc_main_body(u_emb, v_emb, nu2d, nv2d, neg2d, delta,
                  u_id_o, v_id_o, *scr):
    wid = lax.axis_index("s") * 2 + lax.axis_index("c")
    base = wid * _RPW

    idx_u, idx_v, idx_n = scr[0:3]
    ring = scr[3:3 + _NR]
    potc, dbuf = scr[3 + _NR:5 + _NR]
    gsem = scr[5 + _NR:5 + 2 * _NR]
    wsem = scr[5 + 2 * _NR:5 + 3 * _NR]
    dsem = scr[5 + 3 * _NR]

    pltpu.sync_copy(nu2d.at[pl.ds(4 * wid, 4)], idx_u)
    pltpu.sync_copy(nv2d.at[pl.ds(4 * wid, 4)], idx_v)
    pltpu.sync_copy(neg2d.at[pl.ds(16 * wid, 16)], idx_n)

    jobs = []
    for c in range(4):
        for m in range(4):
            jobs.append(("neg", v_emb, idx_n, 4 * c + m, None, 32 * m))
        jobs.append(("plain", u_emb, idx_u, c, u_id_o, base + _CH * c))
        jobs.append(("vblend", v_emb, idx_v, c, v_id_o, base + _CH * c))

    g = [None] * _NR
    w = [None] * _NR
    dcp = [None]

    def pool(src, off):
        @plsc.parallel_loop(0, 32, unroll=2)
        def _row(r):
            for j in range(_D // 16):
                sl = pl.ds(16 * j, 16)
                s = ((src[4 * r, sl] + src[4 * r + 1, sl])
                     + (src[4 * r + 2, sl] + src[4 * r + 3, sl]))
                potc[off + r, sl] = s * 0.25

    def blend(src):
        @plsc.parallel_loop(0, _CH, unroll=4)
        def _row(r):
            for j in range(_D // 16):
                sl = pl.ds(16 * j, 16)
                p = potc[r, sl]
                src[r, sl] = p + dbuf[r, sl] * (src[r, sl] - p)

    for i in range(len(jobs) + _LAG):
        if i < len(jobs):
            s = i % _NR
            if i >= _NR and w[s] is not None:
                w[s].wait()
                w[s] = None
            kind, tbl, idx, row, out, off = jobs[i]
            g[s] = pltpu.async_copy(tbl.at[idx.at[row]], ring[s], gsem[s])
            if kind == "vblend":
                dcp[0] = pltpu.async_copy(delta.at[pl.ds(off, _CH)], dbuf, dsem)
        p = i - _LAG
        if 0 <= p < len(jobs):
            sp = p % _NR
            g[sp].wait()
            kind, tbl, idx, row, out, off = jobs[p]
            if kind == "neg":
                pool(ring[sp], off)
                w[sp] = None
            elif kind == "vblend":
                dcp[0].wait()
                blend(ring[sp])
                w[sp] = pltpu.async_copy(ring[sp], out.at[pl.ds(off, _CH)],
                                         wsem[sp])
            else:
                w[sp] = pltpu.async_copy(ring[sp], out.at[pl.ds(off, _CH)],
                                         wsem[sp])
    for s in range(_NR):
        if w[s] is not None:
            w[s].wait()


def _sc_rev_body(u_rev, v_rev, nu2d, nv2d, u_rev_o, v_rev_o,
                 idx_u, idx_v, r0, r1, r2, r3,
                 gs0, gs1, gs2, gs3, ws0, ws1, ws2, ws3):
    wid = lax.axis_index("s") * 2 + lax.axis_index("c")
    base = wid * _RPW

    ring = (r0, r1, r2, r3)
    gsem = (gs0, gs1, gs2, gs3)
    wsem = (ws0, ws1, ws2, ws3)

    pltpu.sync_copy(nu2d.at[pl.ds(4 * wid, 4)], idx_u)
    pltpu.sync_copy(nv2d.at[pl.ds(4 * wid, 4)], idx_v)

    jobs = []
    for c in range(4):
        jobs.append((u_rev, idx_u, c, u_rev_o, base + _CH * c))
        jobs.append((v_rev, idx_v, c, v_rev_o, base + _CH * c))

    g = [None, None, None, None]
    w = [None, None, None, None]
    for i in range(len(jobs) + 3):
        if i < len(jobs):
            s = i % 4
            if w[s] is not None:
                w[s].wait()
                w[s] = None
            tbl, idx, row, out, off = jobs[i]
            g[s] = pltpu.async_copy(tbl.at[idx.at[row]], ring[s], gsem[s])
        p = i - 3
        if 0 <= p < len(jobs):
            sp = p % 4
            g[sp].wait()
            tbl, idx, row, out, off = jobs[p]
            w[sp] = pltpu.async_copy(ring[sp], out.at[pl.ds(off, _CH)],
                                     wsem[sp])
    for s in range(4):
        if w[s] is not None:
            w[s].wait()


def _sc_gather_main(u_emb, v_emb, nu2d, nv2d, neg2d, delta):
    f32 = jnp.float32
    out = jax.ShapeDtypeStruct((_B, _D), f32)
    run = pl.kernel(
        _sc_main_body,
        mesh=plsc.VectorSubcoreMesh(core_axis_name="c", subcore_axis_name="s"),
        out_type=[out, out],
        scratch_types=(
            [pltpu.VMEM((4, 128), jnp.int32),
             pltpu.VMEM((4, 128), jnp.int32),
             pltpu.VMEM((16, 128), jnp.int32)]
            + [pltpu.VMEM((_CH, _D), f32) for _ in range(_NR)]
            + [pltpu.VMEM((_CH, _D), f32),
               pltpu.VMEM((_CH, _D), f32)]
            + [pltpu.SemaphoreType.DMA for _ in range(2 * _NR + 1)]
        ),
    )
    return run(u_emb, v_emb, nu2d, nv2d, neg2d, delta)


def _sc_gather_rev(u_rev, v_rev, nu2d, nv2d):
    f32 = jnp.float32
    out = jax.ShapeDtypeStruct((_B, _D), f32)
    run = pl.kernel(
        _sc_rev_body,
        mesh=plsc.VectorSubcoreMesh(core_axis_name="c", subcore_axis_name="s"),
        out_type=[out, out],
        scratch_types=(
            [pltpu.VMEM((4, 128), jnp.int32),
             pltpu.VMEM((4, 128), jnp.int32)]
            + [pltpu.VMEM((_CH, _D), f32) for _ in range(4)]
            + [pltpu.SemaphoreType.DMA for _ in range(8)]
        ),
    )
    return run(u_rev, v_rev, nu2d, nv2d)


def _blend_body(vraw, pot, delta, out):
    p = pot[:]
    out[:] = p + delta[:] * (vraw[:] - p)


def _blend(vraw, pot, delta):
    blk = pl.BlockSpec((1024, _D), lambda i: (i, 0))
    return pl.pallas_call(
        _blend_body,
        grid=(_B // 1024,),
        in_specs=[blk, blk, blk],
        out_specs=blk,
        out_shape=jax.ShapeDtypeStruct((_B, _D), jnp.float32),
    )(vraw, pot, delta)


_MBLK = 8192
_NBLK = _B // _MBLK


def _mlp_body(u_ref, v_ref, w1u, w1v, b1, g1, be1, w2, b2, g2, be2,
              w3, b3, g3, be3, wc, bc, out_ref,
              z1s, z2s, z3s, s1, q1, s2, q2, s3, q3):
    eps = 1e-5
    inv_b = 1.0 / _B
    ph = pl.program_id(0)
    b = pl.program_id(1)

    def mm(x, w):
        return jnp.dot(x, w, preferred_element_type=jnp.float32)

    def acc(sref, qref, z):
        @pl.when(b == 0)
        def _():
            sref[:] = jnp.zeros_like(sref)
            qref[:] = jnp.zeros_like(qref)
        sref[:] += jnp.sum(z, axis=0, keepdims=True)
        qref[:] += jnp.sum(z * z, axis=0, keepdims=True)

    def norm(sref, qref, z, g, be):
        mu = sref[:] * inv_b
        var = qref[:] * inv_b - mu * mu
        return g[:] * (z - mu) * lax.rsqrt(var + eps) + be[:]

    @pl.when(ph == 0)
    def _():
        z1 = jnp.maximum(mm(u_ref[:], w1u[:]) + mm(v_ref[:], w1v[:]) + b1[:],
                         0.0)
        z1s[pl.ds(b * _MBLK, _MBLK), :] = z1
        acc(s1, q1, z1)

    @pl.when(ph == 1)
    def _():
        h1 = norm(s1, q1, z1s[pl.ds(b * _MBLK, _MBLK), :], g1, be1)
        z2 = jnp.maximum(mm(h1, w2[:]) + b2[:], 0.0)
        z2s[pl.ds(b * _MBLK, _MBLK), :] = z2
        acc(s2, q2, z2)

    @pl.when(ph == 2)
    def _():
        h2 = norm(s2, q2, z2s[pl.ds(b * _MBLK, _MBLK), :], g2, be2)
        z3 = jnp.maximum(mm(h2, w3[:]) + b3[:], 0.0)
        z3s[pl.ds(b * _MBLK, _MBLK), :] = z3
        acc(s3, q3, z3)

    @pl.when(ph == 3)
    def _():
        h3 = norm(s3, q3, z3s[pl.ds(b * _MBLK, _MBLK), :], g3, be3)
        logit = mm(h3, wc[:]) + bc[:]
        out_ref[:] = 1.0 / (1.0 + jnp.exp(-logit))


def _mlp(u_id, v_id, W1, b1, W2, b2, W3, b3, Wc, bc, g1, be1, g2, be2, g3, be3):
    f32 = jnp.float32
    io_spec = pl.BlockSpec(
        (_MBLK, _D), lambda ph, b: (jnp.where(ph == 0, b, _NBLK - 1), 0))
    full = lambda r, c: pl.BlockSpec((r, c), lambda ph, b: (0, 0))
    return pl.pallas_call(
        _mlp_body,
        grid=(4, _NBLK),
        in_specs=[io_spec, io_spec,
                  full(_D, _D), full(_D, _D), full(1, _D), full(1, _D),
                  full(1, _D), full(_D, _D // 2), full(1, _D // 2),
                  full(1, _D // 2), full(1, _D // 2), full(_D // 2, _D // 4),
                  full(1, _D // 4), full(1, _D // 4), full(1, _D // 4),
                  full(_D // 4, 1), full(1, 1)],
        out_specs=pl.BlockSpec((_MBLK, 1),
                               lambda ph, b: (jnp.where(ph == 3, b, 0), 0)),
        out_shape=jax.ShapeDtypeStruct((_B, 1), f32),
        scratch_shapes=[
            pltpu.VMEM((_B, _D), f32),
            pltpu.VMEM((_B, _D // 2), f32),
            pltpu.VMEM((_B, _D // 4), f32),
            pltpu.VMEM((1, _D), f32), pltpu.VMEM((1, _D), f32),
            pltpu.VMEM((1, _D // 2), f32), pltpu.VMEM((1, _D // 2), f32),
            pltpu.VMEM((1, _D // 4), f32), pltpu.VMEM((1, _D // 4), f32),
        ],
    )(u_id, v_id,
      W1[:, :_D].T, W1[:, _D:].T, b1.reshape(1, -1), g1.reshape(1, -1),
      be1.reshape(1, -1), W2.T, b2.reshape(1, -1), g2.reshape(1, -1),
      be2.reshape(1, -1), W3.T, b3.reshape(1, -1), g3.reshape(1, -1),
      be3.reshape(1, -1), Wc.T, bc.reshape(1, -1))


def kernel(nodes_u, nodes_v, global_protos, inter_nums, u_emb_w, v_emb_w,
           u_rev_w, v_rev_w, W1, b1, W2, b2, W3, b3, Wc, bc,
           g1, be1, g2, be2, g3, be3):
    neg2d, delta = _rng_consts()
    nu2d = nodes_u.astype(jnp.int32).reshape(_B // 128, 128)
    nv2d = nodes_v.astype(jnp.int32).reshape(_B // 128, 128)
    u_id, v_id = _sc_gather_main(u_emb_w, v_emb_w, nu2d, nv2d, neg2d, delta)
    u_review, v_review = _sc_gather_rev(u_rev_w, v_rev_w, nu2d, nv2d)
    pred = _mlp(u_id, v_id, W1, b1, W2, b2, W3, b3, Wc, bc,
                g1, be1, g2, be2, g3, be3)
    return (pred[:, 0], u_id, v_id, u_review, v_review)
